# baseline (speedup 1.0000x reference)
import jax
import jax.numpy as jnp
from jax import lax
from jax.experimental import pallas as pl
from jax.experimental.pallas import tpu as pltpu
from jax.experimental.pallas import tpu_sc as plsc

B = 4096
F = 26
D = 64
L = 16
NC = 2
NS = 16
NW = NC * NS
BPW = B // NW
G = 4
ROWS = F * G
NG = BPW // G
NBUF = 4
NCHUNK = D // L


def _fm_body(xg_hbm, emb_hbm, bias_hbm, w0_hbm, out_hbm,
             idx_a, ebuf, bias_buf, fm_v, w0_v, esems, bsems):
    wid = lax.axis_index("s") * NC + lax.axis_index("c")
    base = wid * BPW

    pltpu.sync_copy(xg_hbm.at[wid], idx_a)
    pltpu.sync_copy(w0_hbm, w0_v)

    lane = lax.broadcasted_iota(jnp.int32, (L,), 0)
    tail_mask = lane < (F - L)

    def start_group(g, b):
        pltpu.make_async_copy(emb_hbm.at[idx_a.at[g]], ebuf.at[b],
                              esems[b]).start()
        pltpu.make_async_copy(bias_hbm.at[idx_a.at[g]],
                              bias_buf.at[pl.ds(g * ROWS, ROWS)],
                              bsems[b]).start()

    def wait_group(g, b):
        pltpu.make_async_copy(emb_hbm.at[idx_a.at[g]], ebuf.at[b],
                              esems[b]).wait()
        pltpu.make_async_copy(bias_hbm.at[idx_a.at[g]],
                              bias_buf.at[pl.ds(g * ROWS, ROWS)],
                              bsems[b]).wait()

    for b in range(NBUF):
        start_group(b, b)

    def body(i, fm_vec):
        for b in range(NBUF):
            g = i * NBUF + b
            wait_group(g, b)
            eb = ebuf.at[b]
            boff = g * ROWS
            for it in range(G):
                fm_acc = jnp.zeros((L,), jnp.float32)
                for c in range(NCHUNK):
                    s = jnp.zeros((L,), jnp.float32)
                    q = jnp.zeros((L,), jnp.float32)
                    for f in range(F):
                        v = eb[it * F + f, pl.ds(c * L, L)]
                        s = s + v
                        q = q + v * v
                    fm_acc = fm_acc + (s * s - q)
                b1 = bias_buf[pl.ds(boff + it * F, L)]
                b2 = bias_buf[pl.ds(boff + it * F + L, L)]
                b2 = jnp.where(tail_mask, b2, 0.0)
                tot = jnp.sum(0.5 * fm_acc + b1 + b2)
                fm_vec = jnp.where(lane == (b * G + it), tot, fm_vec)
            ng = g + NBUF

            @pl.when(ng < NG)
            def _():
                start_group(ng, b)
        fm_v[pl.ds(i * L, L)] = fm_vec + w0_v[...]
        return fm_vec

    lax.fori_loop(0, NG // NBUF, body, jnp.zeros((L,), jnp.float32))

    pltpu.sync_copy(fm_v, out_hbm.at[pl.ds(base, BPW)])


def kernel(x, emb_table, bias_table, w0):
    xg = ((jnp.arange(NW * NG * ROWS, dtype=jnp.uint32) * jnp.uint32(2654435761))
          % jnp.uint32(100000)).astype(jnp.int32).reshape(NW, NG, ROWS)
    bias_flat = jnp.zeros((100000,), jnp.float32)
    w0b = jnp.broadcast_to(w0, (L,))

    mesh = plsc.VectorSubcoreMesh(core_axis_name="c", subcore_axis_name="s")
    run = pl.kernel(
        _fm_body,
        out_type=jax.ShapeDtypeStruct((B,), jnp.float32),
        mesh=mesh,
        compiler_params=pltpu.CompilerParams(needs_layout_passes=False,
                                             use_tc_tiling_on_sc=False),
        scratch_types=[
            pltpu.VMEM((NG, ROWS), jnp.int32),
            pltpu.VMEM((NBUF, ROWS, D), jnp.float32),
            pltpu.VMEM((NG * ROWS + L,), jnp.float32),
            pltpu.VMEM((BPW,), jnp.float32),
            pltpu.VMEM((L,), jnp.float32),
            [pltpu.SemaphoreType.DMA] * NBUF,
            [pltpu.SemaphoreType.DMA] * NBUF,
        ],
    )
    return run(xg, emb_table, bias_flat, w0b)


# speedup vs baseline: 1.0590x; 1.0590x over previous
import jax
import jax.numpy as jnp
from jax import lax
from jax.experimental import pallas as pl
from jax.experimental.pallas import tpu as pltpu
from jax.experimental.pallas import tpu_sc as plsc

B = 4096
F = 26
D = 64
L = 16
NC = 2
NS = 16
NW = NC * NS
BPW = B // NW
G = 4
ROWS = F * G
NG = BPW // G
NBUF = 4
NCHUNK = D // L


def _fm_body(xg_hbm, emb_hbm, bias_hbm, w0_hbm, out_hbm,
             idx_a, ebuf, bias_buf, fm_v, w0_v, esems, bsems):
    wid = lax.axis_index("s") * NC + lax.axis_index("c")
    base = wid * BPW

    pltpu.sync_copy(xg_hbm.at[wid], idx_a)
    pltpu.sync_copy(w0_hbm, w0_v)

    lane = lax.broadcasted_iota(jnp.int32, (L,), 0)
    tail_mask = lane < (F - L)

    def start_group(g, b):
        pltpu.make_async_copy(emb_hbm.at[idx_a.at[g]], ebuf.at[b],
                              esems[b]).start()
        pltpu.make_async_copy(bias_hbm.at[idx_a.at[g]],
                              bias_buf.at[pl.ds(g * ROWS, ROWS)],
                              bsems[b]).start()

    def wait_group(g, b):
        pltpu.make_async_copy(emb_hbm.at[idx_a.at[g]], ebuf.at[b],
                              esems[b]).wait()
        pltpu.make_async_copy(bias_hbm.at[idx_a.at[g]],
                              bias_buf.at[pl.ds(g * ROWS, ROWS)],
                              bsems[b]).wait()

    for b in range(NBUF):
        start_group(b, b)

    def body(i, fm_vec):
        for b in range(NBUF):
            g = i * NBUF + b
            wait_group(g, b)
            eb = ebuf.at[b]
            boff = g * ROWS
            for it in range(G):
                fm_acc = jnp.zeros((L,), jnp.float32)
                for c in range(NCHUNK):
                    s = jnp.zeros((L,), jnp.float32)
                    q = jnp.zeros((L,), jnp.float32)
                    for f in range(F):
                        v = eb[it * F + f, pl.ds(c * L, L)]
                        s = s + v
                        q = q + v * v
                    fm_acc = fm_acc + (s * s - q)
                b1 = bias_buf[pl.ds(boff + it * F, L)]
                b2 = bias_buf[pl.ds(boff + it * F + L, L)]
                b2 = jnp.where(tail_mask, b2, 0.0)
                tot = jnp.sum(0.5 * fm_acc + b1 + b2)
                fm_vec = jnp.where(lane == (b * G + it), tot, fm_vec)
            ng = g + NBUF

            @pl.when(ng < NG)
            def _():
                start_group(ng, b)
        fm_v[pl.ds(i * L, L)] = fm_vec + w0_v[...]
        return fm_vec

    lax.fori_loop(0, NG // NBUF, body, jnp.zeros((L,), jnp.float32))

    pltpu.sync_copy(fm_v, out_hbm.at[pl.ds(base, BPW)])


def kernel(x, emb_table, bias_table, w0):
    xg = x.reshape(NW, NG, ROWS)
    bias_flat = bias_table.reshape(-1)
    w0b = jnp.broadcast_to(w0, (L,))
    emb_wide = jnp.pad(emb_table, ((0, 0), (0, D)))

    mesh = plsc.VectorSubcoreMesh(core_axis_name="c", subcore_axis_name="s")
    run = pl.kernel(
        _fm_body,
        out_type=jax.ShapeDtypeStruct((B,), jnp.float32),
        mesh=mesh,
        compiler_params=pltpu.CompilerParams(needs_layout_passes=False,
                                             use_tc_tiling_on_sc=False),
        scratch_types=[
            pltpu.VMEM((NG, ROWS), jnp.int32),
            pltpu.VMEM((NBUF, ROWS, 2 * D), jnp.float32),
            pltpu.VMEM((NG * ROWS + L,), jnp.float32),
            pltpu.VMEM((BPW,), jnp.float32),
            pltpu.VMEM((L,), jnp.float32),
            [pltpu.SemaphoreType.DMA] * NBUF,
            [pltpu.SemaphoreType.DMA] * NBUF,
        ],
    )
    return run(xg, emb_wide, bias_flat, w0b)
